# baseline (speedup 1.0000x reference)
import functools

import jax
import jax.numpy as jnp
from jax import lax
from jax.experimental import pallas as pl
from jax.experimental.pallas import tpu as pltpu
from jax.experimental.pallas import tpu_sc as plsc

BATCH = 16384
DIM = 64
L = 16
NTAB = 1000

_info = plsc.get_sparse_core_info()
NC, NS = _info.num_cores, _info.num_subcores
NW = NC * NS
B_PER_W = BATCH // NW
CHUNK = 128
N_CHUNKS = B_PER_W // CHUNK


def _body(s_id, r_id, o_id, t_id, ent, rel_flat, tim_flat, out,
          sidx, ridx, oidx, tidx, tab, rt, srow, orow, outv, sem, semt):
    wid = lax.axis_index("s") * NC + lax.axis_index("c")
    base = wid * B_PER_W
    lanes = lax.iota(jnp.int32, L)

    cp_tab = pltpu.async_copy(rel_flat, tab, semt)
    pltpu.sync_copy(s_id.at[pl.ds(base, B_PER_W)], sidx)
    pltpu.sync_copy(r_id.at[pl.ds(base, B_PER_W)], ridx)
    pltpu.sync_copy(o_id.at[pl.ds(base, B_PER_W)], oidx)
    pltpu.sync_copy(t_id.at[pl.ds(base, B_PER_W)], tidx)

    cp_s0 = pltpu.async_copy(ent.at[sidx.at[pl.ds(0, CHUNK)]], srow, sem)
    cp_o0 = pltpu.async_copy(ent.at[oidx.at[pl.ds(0, CHUNK)]], orow, sem)
    cp_tab.wait()

    def rel_group(g, _):
        tv = ridx[pl.ds(g * L, L)] * DIM

        def d_body(d, _):
            rt[d, pl.ds(g * L, L)] = plsc.load_gather(tab, [tv + d])
            return 0

        lax.fori_loop(0, DIM, d_body, 0)
        return 0

    lax.fori_loop(0, B_PER_W // L, rel_group, 0)

    pltpu.sync_copy(tim_flat, tab)

    def tim_group(g, _):
        tv = tidx[pl.ds(g * L, L)] * DIM

        def d_body(d, _):
            rt[d, pl.ds(g * L, L)] += plsc.load_gather(tab, [tv + d])
            return 0

        lax.fori_loop(0, DIM, d_body, 0)
        return 0

    lax.fori_loop(0, B_PER_W // L, tim_group, 0)

    cp_s0.wait()
    cp_o0.wait()

    for c in range(N_CHUNKS):
        cb = c * CHUNK

        def score_group(g, acc_unused):
            rowv = lanes + g * L

            def d_body(d, acc):
                col = jnp.full((L,), 0, jnp.int32) + d
                sv = plsc.load_gather(srow, [rowv, col])
                ov = plsc.load_gather(orow, [rowv, col])
                rtv = rt[d, pl.ds(cb + g * L, L)]
                diff = sv + rtv - ov
                return acc + diff * diff

            acc = lax.fori_loop(0, DIM, d_body, jnp.zeros((L,), jnp.float32))
            seed = jnp.int32(0x5F3759DF) - (plsc.bitcast(acc, jnp.int32) >> 1)
            y = plsc.bitcast(seed, jnp.float32)
            half = acc * jnp.float32(0.5)
            for _i in range(3):
                y = y * (jnp.float32(1.5) - half * y * y)
            outv[pl.ds(cb + g * L, L)] = -(acc * y)
            return 0

        lax.fori_loop(0, CHUNK // L, score_group, 0)

        if c + 1 < N_CHUNKS:
            nb = (c + 1) * CHUNK
            cp_s = pltpu.async_copy(ent.at[sidx.at[pl.ds(nb, CHUNK)]],
                                    srow, sem)
            cp_o = pltpu.async_copy(ent.at[oidx.at[pl.ds(nb, CHUNK)]],
                                    orow, sem)
            cp_s.wait()
            cp_o.wait()

    pltpu.sync_copy(outv, out.at[pl.ds(base, B_PER_W)])


_sc_call = functools.partial(
    pl.kernel,
    mesh=plsc.VectorSubcoreMesh(core_axis_name="c", subcore_axis_name="s"),
    out_type=jax.ShapeDtypeStruct((BATCH,), jnp.float32),
    compiler_params=pltpu.CompilerParams(use_tc_tiling_on_sc=False,
                                         needs_layout_passes=False),
    scratch_types=[
        pltpu.VMEM((B_PER_W,), jnp.int32),
        pltpu.VMEM((B_PER_W,), jnp.int32),
        pltpu.VMEM((B_PER_W,), jnp.int32),
        pltpu.VMEM((B_PER_W,), jnp.int32),
        pltpu.VMEM((NTAB * DIM,), jnp.float32),
        pltpu.VMEM((DIM, B_PER_W), jnp.float32),
        pltpu.VMEM((CHUNK, DIM), jnp.float32),
        pltpu.VMEM((CHUNK, DIM), jnp.float32),
        pltpu.VMEM((B_PER_W,), jnp.float32),
        pltpu.SemaphoreType.DMA,
        pltpu.SemaphoreType.DMA,
    ],
)(_body)


def kernel(s_id, r_id, o_id, t_id, entities, relations, times):
    return _sc_call(s_id.astype(jnp.int32), r_id.astype(jnp.int32),
                    o_id.astype(jnp.int32), t_id.astype(jnp.int32),
                    entities, relations.reshape(-1), times.reshape(-1))
